# baseline (speedup 1.0000x reference)
import functools

import jax
import jax.numpy as jnp
from jax import lax
from jax.experimental import pallas as pl
from jax.experimental.pallas import tpu as pltpu
from jax.experimental.pallas import tpu_sc as plsc

D_MODEL = 768
D_HIDDEN = 2048
NUM_EXPERTS = 64
N_TOK = 2048
TILE = 64
P_TILES = 95
PADDED = P_TILES * TILE
DUMMY = N_TOK

SC_CORES = 2
SC_SUBCORES = 16
SC_WORKERS = SC_CORES * SC_SUBCORES
TOK_PER_W = N_TOK // SC_WORKERS
SLOT_PER_W = PADDED // SC_WORKERS
SC_CHUNK = 64


def _router_body(x_ref, w_ref, b_ref, slot_ref, prob_ref, te_ref, loss_ref):
    xf = x_ref[...]
    logits = jnp.dot(xf, w_ref[...], preferred_element_type=jnp.float32)
    logits = logits + b_ref[...]
    m = jnp.max(logits, axis=1, keepdims=True)
    ex = jnp.exp(logits - m)
    s = jnp.sum(ex, axis=1, keepdims=True)
    probs = ex / s

    pm = jnp.sum(probs, axis=0, keepdims=True) * (1.0 / N_TOK)
    loss_ref[...] = jnp.sum(pm * pm, axis=1, keepdims=True) * NUM_EXPERTS

    iota_e = lax.broadcasted_iota(jnp.int32, (N_TOK, NUM_EXPERTS), 1)
    idx = jnp.min(jnp.where(logits == m, iota_e, NUM_EXPERTS),
                  axis=1, keepdims=True)
    onehot = (iota_e == idx).astype(jnp.float32)

    counts = jnp.sum(onehot, axis=0, keepdims=True)
    tiles_per = jnp.ceil(counts * (1.0 / TILE))
    tri = (lax.broadcasted_iota(jnp.int32, (NUM_EXPERTS, NUM_EXPERTS), 0)
           <= lax.broadcasted_iota(jnp.int32, (NUM_EXPERTS, NUM_EXPERTS), 1)
           ).astype(jnp.float32)
    end = jnp.dot(tiles_per, tri, preferred_element_type=jnp.float32)
    start = end - tiles_per

    lower = (lax.broadcasted_iota(jnp.int32, (N_TOK, N_TOK), 1)
             < lax.broadcasted_iota(jnp.int32, (N_TOK, N_TOK), 0)
             ).astype(jnp.float32)
    ranks = jnp.dot(lower, onehot, preferred_element_type=jnp.float32)
    rank_t = jnp.sum(ranks * onehot, axis=1, keepdims=True)
    start_t = jnp.sum(onehot * start, axis=1, keepdims=True)
    slot_ref[...] = (start_t * TILE + rank_t).astype(jnp.int32)
    prob_ref[...] = 1.0 / s

    end_i = end.astype(jnp.int32)
    pcol = lax.broadcasted_iota(jnp.int32, (P_TILES, NUM_EXPERTS), 0)
    te = jnp.sum((end_i <= pcol).astype(jnp.int32), axis=1, keepdims=True)
    te_ref[...] = jnp.minimum(te, NUM_EXPERTS - 1)


def _router(x_flat, router_W, router_b):
    return pl.pallas_call(
        _router_body,
        out_shape=(
            jax.ShapeDtypeStruct((N_TOK, 1), jnp.int32),
            jax.ShapeDtypeStruct((N_TOK, 1), jnp.float32),
            jax.ShapeDtypeStruct((P_TILES, 1), jnp.int32),
            jax.ShapeDtypeStruct((1, 1), jnp.float32),
        ),
        compiler_params=pltpu.CompilerParams(
            vmem_limit_bytes=100 * 1024 * 1024),
    )(x_flat, router_W, router_b.reshape(1, NUM_EXPERTS))


@functools.lru_cache(maxsize=None)
def _sc_kernels():
    mesh = plsc.VectorSubcoreMesh(core_axis_name="c", subcore_axis_name="s",
                                  num_cores=SC_CORES,
                                  num_subcores=SC_SUBCORES)

    @functools.partial(
        pl.kernel,
        mesh=mesh,
        out_type=(
            jax.ShapeDtypeStruct((PADDED, D_MODEL), jnp.float32),
            jax.ShapeDtypeStruct((PADDED,), jnp.float32),
        ),
        scratch_types=[
            pltpu.VMEM((TOK_PER_W,), jnp.int32),
            pltpu.VMEM((TOK_PER_W, D_MODEL), jnp.float32),
            pltpu.VMEM((N_TOK,), jnp.int32),
            pltpu.VMEM((N_TOK,), jnp.float32),
            pltpu.VMEM((PADDED,), jnp.float32),
            pltpu.SemaphoreType.DMA,
        ],
        compiler_params=pltpu.CompilerParams(needs_layout_passes=False),
    )
    def _dispatch(x_hbm, slot_hbm, prob_hbm, xs_hbm, pr_hbm,
                  slot_v, x_v, slots_all, prob_all, pr_v, sem):
        wid = lax.axis_index("s") * SC_CORES + lax.axis_index("c")
        base = wid * TOK_PER_W
        pltpu.sync_copy(slot_hbm.at[pl.ds(base, TOK_PER_W)], slot_v)
        pltpu.sync_copy(x_hbm.at[pl.ds(base, TOK_PER_W)], x_v)
        pltpu.async_copy(x_v, xs_hbm.at[slot_v], sem).wait()

        @pl.when(wid == 0)
        def _():
            pltpu.sync_copy(slot_hbm, slots_all)
            pltpu.sync_copy(prob_hbm, prob_all)

            def scat_body(i, carry):
                sl = slots_all[pl.ds(i * 16, 16)]
                pv = prob_all[pl.ds(i * 16, 16)]
                plsc.store_scatter(pr_v, [sl], pv)
                return carry

            lax.fori_loop(0, N_TOK // 16, scat_body, 0)
            pltpu.sync_copy(pr_v, pr_hbm)

    @functools.partial(
        pl.kernel,
        mesh=mesh,
        out_type=jax.ShapeDtypeStruct((N_TOK, D_MODEL), jnp.float32),
        scratch_types=[
            pltpu.VMEM((TOK_PER_W,), jnp.int32),
            pltpu.VMEM((TOK_PER_W, D_MODEL), jnp.float32),
            pltpu.SemaphoreType.DMA,
        ],
    )
    def _combine(y_hbm, slot_hbm, out_hbm, idx_v, y_v, sem):
        wid = lax.axis_index("s") * SC_CORES + lax.axis_index("c")
        base = wid * TOK_PER_W
        pltpu.sync_copy(slot_hbm.at[pl.ds(base, TOK_PER_W)], idx_v)
        pltpu.async_copy(y_hbm.at[idx_v], y_v, sem).wait()
        pltpu.sync_copy(y_v, out_hbm.at[pl.ds(base, TOK_PER_W)])

    return _dispatch, _combine


def _ffn_body(te_ref, x_ref, w1_ref, b1_ref, w2_ref, b2_ref, wo_ref, ob_ref,
              p_ref, y_ref):
    xb = x_ref[...]
    h1 = jnp.dot(xb, w1_ref[0], preferred_element_type=jnp.float32)
    h1 = h1 + b1_ref[0]
    h2 = jnp.dot(xb, w2_ref[0], preferred_element_type=jnp.float32)
    h2 = h2 + b2_ref[0]
    h = h1 * (h2 * jax.nn.sigmoid(h2))
    eo = jnp.dot(h, wo_ref[0], preferred_element_type=jnp.float32)
    eo = eo + ob_ref[0]
    y_ref[...] = eo * p_ref[...]


def _ffn(tile_expert, x_sorted, w1_W, w1_b, w2_W, w2_b, out_W, out_b, prob2d):
    grid_spec = pltpu.PrefetchScalarGridSpec(
        num_scalar_prefetch=1,
        grid=(P_TILES,),
        in_specs=[
            pl.BlockSpec((TILE, D_MODEL), lambda i, te: (i, 0)),
            pl.BlockSpec((1, D_MODEL, D_HIDDEN), lambda i, te: (te[i], 0, 0)),
            pl.BlockSpec((1, 1, D_HIDDEN), lambda i, te: (te[i], 0, 0)),
            pl.BlockSpec((1, D_MODEL, D_HIDDEN), lambda i, te: (te[i], 0, 0)),
            pl.BlockSpec((1, 1, D_HIDDEN), lambda i, te: (te[i], 0, 0)),
            pl.BlockSpec((1, D_HIDDEN, D_MODEL), lambda i, te: (te[i], 0, 0)),
            pl.BlockSpec((1, 1, D_MODEL), lambda i, te: (te[i], 0, 0)),
            pl.BlockSpec((TILE, 1), lambda i, te: (i, 0)),
        ],
        out_specs=pl.BlockSpec((TILE, D_MODEL), lambda i, te: (i, 0)),
    )
    return pl.pallas_call(
        _ffn_body,
        grid_spec=grid_spec,
        out_shape=jax.ShapeDtypeStruct((PADDED, D_MODEL), jnp.float32),
        compiler_params=pltpu.CompilerParams(
            vmem_limit_bytes=100 * 1024 * 1024),
    )(tile_expert, x_sorted,
      w1_W, w1_b.reshape(NUM_EXPERTS, 1, D_HIDDEN),
      w2_W, w2_b.reshape(NUM_EXPERTS, 1, D_HIDDEN),
      out_W, out_b.reshape(NUM_EXPERTS, 1, D_MODEL), prob2d)


def kernel(x, router_W, router_b, w1_W, w1_b, w2_W, w2_b, out_W, out_b):
    B, S, D = x.shape
    x_flat = x.reshape(-1, D)
    slot2d, prob2d, te2d, loss2d = _router(x_flat, router_W, router_b)
    slot = slot2d.reshape(-1)
    prob = prob2d.reshape(-1)
    tile_expert = te2d.reshape(-1)
    _dispatch, _combine = _sc_kernels()
    x_sorted, prob_sorted = _dispatch(x_flat, slot, prob)
    y = _ffn(tile_expert, x_sorted, w1_W, w1_b, w2_W, w2_b, out_W, out_b,
             prob_sorted.reshape(PADDED, 1))
    outp = _combine(y, slot)
    final = outp.reshape(B, S, D)
    return final, loss2d[0, 0]


# speedup vs baseline: 1.0934x; 1.0934x over previous
import functools

import jax
import jax.numpy as jnp
from jax import lax
from jax.experimental import pallas as pl
from jax.experimental.pallas import tpu as pltpu
from jax.experimental.pallas import tpu_sc as plsc

D_MODEL = 768
D_HIDDEN = 2048
NUM_EXPERTS = 64
N_TOK = 2048
TILE = 128
P_TILES = 80
PADDED = P_TILES * TILE
DUMMY = N_TOK

SC_CORES = 2
SC_SUBCORES = 16
SC_WORKERS = SC_CORES * SC_SUBCORES
TOK_PER_W = N_TOK // SC_WORKERS
SLOT_PER_W = PADDED // SC_WORKERS
SC_CHUNK = 64


def _router_body(x_ref, w_ref, b_ref, slot_ref, prob_ref, te_ref, tv_ref,
                 loss_ref):
    xf = x_ref[...]
    logits = jnp.dot(xf, w_ref[...], preferred_element_type=jnp.float32)
    logits = logits + b_ref[...]
    m = jnp.max(logits, axis=1, keepdims=True)
    ex = jnp.exp(logits - m)
    s = jnp.sum(ex, axis=1, keepdims=True)
    probs = ex / s

    pm = jnp.sum(probs, axis=0, keepdims=True) * (1.0 / N_TOK)
    loss_ref[...] = jnp.sum(pm * pm, axis=1, keepdims=True) * NUM_EXPERTS

    iota_e = lax.broadcasted_iota(jnp.int32, (N_TOK, NUM_EXPERTS), 1)
    idx = jnp.min(jnp.where(logits == m, iota_e, NUM_EXPERTS),
                  axis=1, keepdims=True)
    onehot = (iota_e == idx).astype(jnp.float32)

    counts = jnp.sum(onehot, axis=0, keepdims=True)
    tiles_per = jnp.ceil(counts * (1.0 / TILE))
    tri = (lax.broadcasted_iota(jnp.int32, (NUM_EXPERTS, NUM_EXPERTS), 0)
           <= lax.broadcasted_iota(jnp.int32, (NUM_EXPERTS, NUM_EXPERTS), 1)
           ).astype(jnp.float32)
    end = jnp.dot(tiles_per, tri, preferred_element_type=jnp.float32)
    start = end - tiles_per

    lower = (lax.broadcasted_iota(jnp.int32, (N_TOK, N_TOK), 1)
             < lax.broadcasted_iota(jnp.int32, (N_TOK, N_TOK), 0)
             ).astype(jnp.float32)
    ranks = jnp.dot(lower, onehot, preferred_element_type=jnp.float32)
    rank_t = jnp.sum(ranks * onehot, axis=1, keepdims=True)
    start_t = jnp.sum(onehot * start, axis=1, keepdims=True)
    slot_ref[...] = (start_t * TILE + rank_t).astype(jnp.int32)
    prob_ref[...] = 1.0 / s

    end_i = end.astype(jnp.int32)
    pcol = lax.broadcasted_iota(jnp.int32, (P_TILES, NUM_EXPERTS), 0)
    te = jnp.sum((end_i <= pcol).astype(jnp.int32), axis=1, keepdims=True)
    te_ref[...] = jnp.minimum(te, NUM_EXPERTS - 1)
    total2d = jnp.sum(tiles_per, axis=1, keepdims=True)
    prow = lax.broadcasted_iota(jnp.int32, (P_TILES, 1), 0)
    tv_ref[...] = (prow.astype(jnp.float32) < total2d).astype(jnp.int32)


def _router(x_flat, router_W, router_b):
    return pl.pallas_call(
        _router_body,
        out_shape=(
            jax.ShapeDtypeStruct((N_TOK, 1), jnp.int32),
            jax.ShapeDtypeStruct((N_TOK, 1), jnp.float32),
            jax.ShapeDtypeStruct((P_TILES, 1), jnp.int32),
            jax.ShapeDtypeStruct((P_TILES, 1), jnp.int32),
            jax.ShapeDtypeStruct((1, 1), jnp.float32),
        ),
        compiler_params=pltpu.CompilerParams(
            vmem_limit_bytes=100 * 1024 * 1024),
    )(x_flat, router_W, router_b.reshape(1, NUM_EXPERTS))


@functools.lru_cache(maxsize=None)
def _sc_kernels():
    mesh = plsc.VectorSubcoreMesh(core_axis_name="c", subcore_axis_name="s",
                                  num_cores=SC_CORES,
                                  num_subcores=SC_SUBCORES)

    @functools.partial(
        pl.kernel,
        mesh=mesh,
        out_type=(
            jax.ShapeDtypeStruct((PADDED, D_MODEL), jnp.float32),
            jax.ShapeDtypeStruct((PADDED,), jnp.float32),
        ),
        scratch_types=[
            pltpu.VMEM((TOK_PER_W,), jnp.int32),
            pltpu.VMEM((TOK_PER_W, D_MODEL), jnp.float32),
            pltpu.VMEM((N_TOK,), jnp.int32),
            pltpu.VMEM((N_TOK,), jnp.float32),
            pltpu.VMEM((PADDED,), jnp.float32),
            pltpu.SemaphoreType.DMA,
        ],
        compiler_params=pltpu.CompilerParams(needs_layout_passes=False),
    )
    def _dispatch(x_hbm, slot_hbm, prob_hbm, xs_hbm, pr_hbm,
                  slot_v, x_v, slots_all, prob_all, pr_v, sem):
        wid = lax.axis_index("s") * SC_CORES + lax.axis_index("c")
        base = wid * TOK_PER_W
        pltpu.sync_copy(slot_hbm.at[pl.ds(base, TOK_PER_W)], slot_v)
        pltpu.sync_copy(x_hbm.at[pl.ds(base, TOK_PER_W)], x_v)
        pltpu.async_copy(x_v, xs_hbm.at[slot_v], sem).wait()

        @pl.when(wid == 0)
        def _():
            pltpu.sync_copy(slot_hbm, slots_all)
            pltpu.sync_copy(prob_hbm, prob_all)

            def scat_body(i, carry):
                sl = slots_all[pl.ds(i * 16, 16)]
                pv = prob_all[pl.ds(i * 16, 16)]
                plsc.store_scatter(pr_v, [sl], pv)
                return carry

            lax.fori_loop(0, N_TOK // 16, scat_body, 0)
            pltpu.sync_copy(pr_v, pr_hbm)

    @functools.partial(
        pl.kernel,
        mesh=mesh,
        out_type=jax.ShapeDtypeStruct((N_TOK, D_MODEL), jnp.float32),
        scratch_types=[
            pltpu.VMEM((TOK_PER_W,), jnp.int32),
            pltpu.VMEM((TOK_PER_W, D_MODEL), jnp.float32),
            pltpu.SemaphoreType.DMA,
        ],
    )
    def _combine(y_hbm, slot_hbm, out_hbm, idx_v, y_v, sem):
        wid = lax.axis_index("s") * SC_CORES + lax.axis_index("c")
        base = wid * TOK_PER_W
        pltpu.sync_copy(slot_hbm.at[pl.ds(base, TOK_PER_W)], idx_v)
        pltpu.async_copy(y_hbm.at[idx_v], y_v, sem).wait()
        pltpu.sync_copy(y_v, out_hbm.at[pl.ds(base, TOK_PER_W)])

    return _dispatch, _combine


def _ffn_body(te_ref, tv_ref, x_ref, w1_ref, b1_ref, w2_ref, b2_ref, wo_ref,
              ob_ref, p_ref, y_ref):
    @pl.when(tv_ref[pl.program_id(0)] == 1)
    def _():
        xb = x_ref[...]
        h1 = jnp.dot(xb, w1_ref[0], preferred_element_type=jnp.float32)
        h1 = h1 + b1_ref[0]
        h2 = jnp.dot(xb, w2_ref[0], preferred_element_type=jnp.float32)
        h2 = h2 + b2_ref[0]
        h = h1 * (h2 * jax.nn.sigmoid(h2))
        eo = jnp.dot(h, wo_ref[0], preferred_element_type=jnp.float32)
        eo = eo + ob_ref[0]
        y_ref[...] = eo * p_ref[...]


def _ffn(tile_expert, tile_valid, x_sorted, w1_W, w1_b, w2_W, w2_b, out_W,
         out_b, prob2d):
    grid_spec = pltpu.PrefetchScalarGridSpec(
        num_scalar_prefetch=2,
        grid=(P_TILES,),
        in_specs=[
            pl.BlockSpec((TILE, D_MODEL), lambda i, te, tv: (i, 0)),
            pl.BlockSpec((1, D_MODEL, D_HIDDEN),
                         lambda i, te, tv: (te[i], 0, 0)),
            pl.BlockSpec((1, 1, D_HIDDEN), lambda i, te, tv: (te[i], 0, 0)),
            pl.BlockSpec((1, D_MODEL, D_HIDDEN),
                         lambda i, te, tv: (te[i], 0, 0)),
            pl.BlockSpec((1, 1, D_HIDDEN), lambda i, te, tv: (te[i], 0, 0)),
            pl.BlockSpec((1, D_HIDDEN, D_MODEL),
                         lambda i, te, tv: (te[i], 0, 0)),
            pl.BlockSpec((1, 1, D_MODEL), lambda i, te, tv: (te[i], 0, 0)),
            pl.BlockSpec((TILE, 1), lambda i, te, tv: (i, 0)),
        ],
        out_specs=pl.BlockSpec((TILE, D_MODEL), lambda i, te, tv: (i, 0)),
    )
    return pl.pallas_call(
        _ffn_body,
        grid_spec=grid_spec,
        out_shape=jax.ShapeDtypeStruct((PADDED, D_MODEL), jnp.float32),
        compiler_params=pltpu.CompilerParams(
            vmem_limit_bytes=100 * 1024 * 1024),
    )(tile_expert, tile_valid, x_sorted,
      w1_W, w1_b.reshape(NUM_EXPERTS, 1, D_HIDDEN),
      w2_W, w2_b.reshape(NUM_EXPERTS, 1, D_HIDDEN),
      out_W, out_b.reshape(NUM_EXPERTS, 1, D_MODEL), prob2d)


def kernel(x, router_W, router_b, w1_W, w1_b, w2_W, w2_b, out_W, out_b):
    B, S, D = x.shape
    x_flat = x.reshape(-1, D)
    slot2d, prob2d, te2d, tv2d, loss2d = _router(x_flat, router_W, router_b)
    slot = slot2d.reshape(-1)
    prob = prob2d.reshape(-1)
    tile_expert = te2d.reshape(-1)
    tile_valid = tv2d.reshape(-1)
    _dispatch, _combine = _sc_kernels()
    x_sorted, prob_sorted = _dispatch(x_flat, slot, prob)
    y = _ffn(tile_expert, tile_valid, x_sorted, w1_W, w1_b, w2_W, w2_b,
             out_W, out_b, prob_sorted.reshape(PADDED, 1))
    outp = _combine(y, slot)
    final = outp.reshape(B, S, D)
    return final, loss2d[0, 0]


# speedup vs baseline: 1.1083x; 1.0136x over previous
import functools

import jax
import jax.numpy as jnp
from jax import lax
from jax.experimental import pallas as pl
from jax.experimental.pallas import tpu as pltpu
from jax.experimental.pallas import tpu_sc as plsc

D_MODEL = 768
D_HIDDEN = 2048
NUM_EXPERTS = 64
N_TOK = 2048
TILE = 128
P_TILES = 80
PADDED = P_TILES * TILE
DUMMY = N_TOK

SC_CORES = 2
SC_SUBCORES = 16
SC_WORKERS = SC_CORES * SC_SUBCORES
TOK_PER_W = N_TOK // SC_WORKERS
SLOT_PER_W = PADDED // SC_WORKERS
SC_CHUNK = 64


def _router_body(x_ref, w_ref, b_ref, slot_ref, prob_ref, te_ref, tv_ref,
                 ym_ref, loss_ref):
    xf = x_ref[...]
    logits = jnp.dot(xf, w_ref[...], preferred_element_type=jnp.float32)
    logits = logits + b_ref[...]
    m = jnp.max(logits, axis=1, keepdims=True)
    ex = jnp.exp(logits - m)
    s = jnp.sum(ex, axis=1, keepdims=True)
    probs = ex / s

    pm = jnp.sum(probs, axis=0, keepdims=True) * (1.0 / N_TOK)
    loss_ref[...] = jnp.sum(pm * pm, axis=1, keepdims=True) * NUM_EXPERTS

    iota_e = lax.broadcasted_iota(jnp.int32, (N_TOK, NUM_EXPERTS), 1)
    idx = jnp.min(jnp.where(logits == m, iota_e, NUM_EXPERTS),
                  axis=1, keepdims=True)
    onehot = (iota_e == idx).astype(jnp.float32)

    counts = jnp.sum(onehot, axis=0, keepdims=True)
    tiles_per = jnp.ceil(counts * (1.0 / TILE))
    tri = (lax.broadcasted_iota(jnp.int32, (NUM_EXPERTS, NUM_EXPERTS), 0)
           <= lax.broadcasted_iota(jnp.int32, (NUM_EXPERTS, NUM_EXPERTS), 1)
           ).astype(jnp.float32)
    end = jnp.dot(tiles_per, tri, preferred_element_type=jnp.float32)
    start = end - tiles_per

    lower = (lax.broadcasted_iota(jnp.int32, (N_TOK, N_TOK), 1)
             < lax.broadcasted_iota(jnp.int32, (N_TOK, N_TOK), 0)
             ).astype(jnp.float32)
    ranks = jnp.dot(lower, onehot, preferred_element_type=jnp.float32)
    rank_t = jnp.sum(ranks * onehot, axis=1, keepdims=True)
    start_t = jnp.sum(onehot * start, axis=1, keepdims=True)
    slot_ref[...] = (start_t * TILE + rank_t).astype(jnp.int32)
    prob_ref[...] = 1.0 / s

    end_i = end.astype(jnp.int32)
    total2d = jnp.sum(tiles_per, axis=1, keepdims=True)
    total_i = total2d.astype(jnp.int32)
    pcol = lax.broadcasted_iota(jnp.int32, (P_TILES, NUM_EXPERTS), 0)
    pclamp = jnp.minimum(pcol, total_i - 1)
    te = jnp.sum((end_i <= pclamp).astype(jnp.int32), axis=1, keepdims=True)
    te_ref[...] = jnp.minimum(te, NUM_EXPERTS - 1)
    prow = lax.broadcasted_iota(jnp.int32, (P_TILES, 1), 0)
    ym_ref[...] = jnp.minimum(prow, total_i - 1)
    tv_ref[...] = (prow < total_i).astype(jnp.int32)


def _router(x_flat, router_W, router_b):
    return pl.pallas_call(
        _router_body,
        out_shape=(
            jax.ShapeDtypeStruct((N_TOK, 1), jnp.int32),
            jax.ShapeDtypeStruct((N_TOK, 1), jnp.float32),
            jax.ShapeDtypeStruct((P_TILES, 1), jnp.int32),
            jax.ShapeDtypeStruct((P_TILES, 1), jnp.int32),
            jax.ShapeDtypeStruct((P_TILES, 1), jnp.int32),
            jax.ShapeDtypeStruct((1, 1), jnp.float32),
        ),
        compiler_params=pltpu.CompilerParams(
            vmem_limit_bytes=100 * 1024 * 1024),
    )(x_flat, router_W, router_b.reshape(1, NUM_EXPERTS))


@functools.lru_cache(maxsize=None)
def _sc_kernels():
    mesh = plsc.VectorSubcoreMesh(core_axis_name="c", subcore_axis_name="s",
                                  num_cores=SC_CORES,
                                  num_subcores=SC_SUBCORES)

    @functools.partial(
        pl.kernel,
        mesh=mesh,
        out_type=(
            jax.ShapeDtypeStruct((PADDED, D_MODEL), jnp.float32),
            jax.ShapeDtypeStruct((PADDED,), jnp.float32),
        ),
        scratch_types=[
            pltpu.VMEM((TOK_PER_W,), jnp.int32),
            pltpu.VMEM((TOK_PER_W, D_MODEL), jnp.float32),
            pltpu.VMEM((N_TOK,), jnp.int32),
            pltpu.VMEM((N_TOK,), jnp.float32),
            pltpu.VMEM((PADDED,), jnp.float32),
            pltpu.SemaphoreType.DMA,
        ],
        compiler_params=pltpu.CompilerParams(needs_layout_passes=False),
    )
    def _dispatch(x_hbm, slot_hbm, prob_hbm, xs_hbm, pr_hbm,
                  slot_v, x_v, slots_all, prob_all, pr_v, sem):
        wid = lax.axis_index("s") * SC_CORES + lax.axis_index("c")
        base = wid * TOK_PER_W
        pltpu.sync_copy(slot_hbm.at[pl.ds(base, TOK_PER_W)], slot_v)
        pltpu.sync_copy(x_hbm.at[pl.ds(base, TOK_PER_W)], x_v)
        pltpu.async_copy(x_v, xs_hbm.at[slot_v], sem).wait()

        @pl.when(wid == 0)
        def _():
            pltpu.sync_copy(slot_hbm, slots_all)
            pltpu.sync_copy(prob_hbm, prob_all)

            def scat_body(i, carry):
                sl = slots_all[pl.ds(i * 16, 16)]
                pv = prob_all[pl.ds(i * 16, 16)]
                plsc.store_scatter(pr_v, [sl], pv)
                return carry

            lax.fori_loop(0, N_TOK // 16, scat_body, 0)
            pltpu.sync_copy(pr_v, pr_hbm)

    @functools.partial(
        pl.kernel,
        mesh=mesh,
        out_type=jax.ShapeDtypeStruct((N_TOK, D_MODEL), jnp.float32),
        scratch_types=[
            pltpu.VMEM((TOK_PER_W,), jnp.int32),
            pltpu.VMEM((TOK_PER_W, D_MODEL), jnp.float32),
            pltpu.SemaphoreType.DMA,
        ],
    )
    def _combine(y_hbm, slot_hbm, out_hbm, idx_v, y_v, sem):
        wid = lax.axis_index("s") * SC_CORES + lax.axis_index("c")
        base = wid * TOK_PER_W
        pltpu.sync_copy(slot_hbm.at[pl.ds(base, TOK_PER_W)], idx_v)
        pltpu.async_copy(y_hbm.at[idx_v], y_v, sem).wait()
        pltpu.sync_copy(y_v, out_hbm.at[pl.ds(base, TOK_PER_W)])

    return _dispatch, _combine


def _ffn_body(te_ref, tv_ref, ym_ref, x_ref, w1_ref, b1_ref, w2_ref, b2_ref,
              wo_ref, ob_ref, p_ref, y_ref):
    @pl.when(tv_ref[pl.program_id(0)] == 1)
    def _():
        xb = x_ref[...]
        h1 = jnp.dot(xb, w1_ref[0], preferred_element_type=jnp.float32)
        h1 = h1 + b1_ref[0]
        h2 = jnp.dot(xb, w2_ref[0], preferred_element_type=jnp.float32)
        h2 = h2 + b2_ref[0]
        h = h1 * (h2 * jax.nn.sigmoid(h2))
        eo = jnp.dot(h, wo_ref[0], preferred_element_type=jnp.float32)
        eo = eo + ob_ref[0]
        y_ref[...] = eo * p_ref[...]


def _ffn(tile_expert, tile_valid, tile_ymap, x_sorted, w1_W, w1_b, w2_W, w2_b,
         out_W, out_b, prob2d):
    grid_spec = pltpu.PrefetchScalarGridSpec(
        num_scalar_prefetch=3,
        grid=(P_TILES,),
        in_specs=[
            pl.BlockSpec((TILE, D_MODEL), lambda i, te, tv, ym: (ym[i], 0)),
            pl.BlockSpec((1, D_MODEL, D_HIDDEN),
                         lambda i, te, tv, ym: (te[i], 0, 0)),
            pl.BlockSpec((1, 1, D_HIDDEN),
                         lambda i, te, tv, ym: (te[i], 0, 0)),
            pl.BlockSpec((1, D_MODEL, D_HIDDEN),
                         lambda i, te, tv, ym: (te[i], 0, 0)),
            pl.BlockSpec((1, 1, D_HIDDEN),
                         lambda i, te, tv, ym: (te[i], 0, 0)),
            pl.BlockSpec((1, D_HIDDEN, D_MODEL),
                         lambda i, te, tv, ym: (te[i], 0, 0)),
            pl.BlockSpec((1, 1, D_MODEL),
                         lambda i, te, tv, ym: (te[i], 0, 0)),
            pl.BlockSpec((TILE, 1), lambda i, te, tv, ym: (ym[i], 0)),
        ],
        out_specs=pl.BlockSpec((TILE, D_MODEL),
                               lambda i, te, tv, ym: (ym[i], 0)),
    )
    return pl.pallas_call(
        _ffn_body,
        grid_spec=grid_spec,
        out_shape=jax.ShapeDtypeStruct((PADDED, D_MODEL), jnp.float32),
        compiler_params=pltpu.CompilerParams(
            vmem_limit_bytes=100 * 1024 * 1024),
    )(tile_expert, tile_valid, tile_ymap, x_sorted,
      w1_W, w1_b.reshape(NUM_EXPERTS, 1, D_HIDDEN),
      w2_W, w2_b.reshape(NUM_EXPERTS, 1, D_HIDDEN),
      out_W, out_b.reshape(NUM_EXPERTS, 1, D_MODEL), prob2d)


def kernel(x, router_W, router_b, w1_W, w1_b, w2_W, w2_b, out_W, out_b):
    B, S, D = x.shape
    x_flat = x.reshape(-1, D)
    slot2d, prob2d, te2d, tv2d, ym2d, loss2d = _router(x_flat, router_W,
                                                       router_b)
    slot = slot2d.reshape(-1)
    prob = prob2d.reshape(-1)
    tile_expert = te2d.reshape(-1)
    tile_valid = tv2d.reshape(-1)
    tile_ymap = ym2d.reshape(-1)
    _dispatch, _combine = _sc_kernels()
    x_sorted, prob_sorted = _dispatch(x_flat, slot, prob)
    y = _ffn(tile_expert, tile_valid, tile_ymap, x_sorted, w1_W, w1_b, w2_W,
             w2_b, out_W, out_b, prob_sorted.reshape(PADDED, 1))
    outp = _combine(y, slot)
    final = outp.reshape(B, S, D)
    return final, loss2d[0, 0]
